# baseline (speedup 1.0000x reference)
import functools

import jax
import jax.numpy as jnp
from jax import lax
from jax.experimental import pallas as pl
from jax.experimental.pallas import tpu as pltpu
from jax.experimental.pallas import tpu_sc as plsc

_NP = 10240
_D = 128
_HD = _D // 2
_NC = 2
_NS = 16
_NW = _NC * _NS
_CHUNK = 128
_BLK = 256


def _make_propagate(nc_a, nc_b):
    nchunk = max(nc_a, nc_b)
    mesh = plsc.VectorSubcoreMesh(core_axis_name="c", subcore_axis_name="s")

    @functools.partial(
        pl.kernel,
        mesh=mesh,
        out_type=jax.ShapeDtypeStruct((_NC, _NP, _D), jnp.float32),
        scratch_types=[
            pltpu.VMEM((nchunk, _CHUNK), jnp.int32),
            pltpu.VMEM((nchunk, _CHUNK), jnp.int32),
            pltpu.VMEM((_CHUNK, _D), jnp.float32),
            pltpu.VMEM_SHARED((_NP, _D), jnp.float32),
            pltpu.SemaphoreType.DMA,
        ],
    )
    def prop(feat_hbm, src_hbm, dst_hbm, out_hbm, src_v, dst_v, rows_v, acc,
             sem):
        c = lax.axis_index("c")
        s = lax.axis_index("s")
        wid = s * _NC + c
        nc_w = jnp.where(c == 0, nc_a, nc_b)
        zero = jnp.zeros((16,), jnp.float32)

        def zrow(i, _):
            for k in range(_D // 16):
                rows_v[i, pl.ds(k * 16, 16)] = zero
            return 0

        lax.fori_loop(0, _CHUNK, zrow, 0)
        rpt = _NP // _NS
        nb = rpt // _CHUNK

        def zacc(k, _):
            pltpu.sync_copy(rows_v, acc.at[pl.ds(s * rpt + k * _CHUNK, _CHUNK)])
            return 0

        lax.fori_loop(0, nb, zacc, 0)
        pltpu.sync_copy(src_hbm.at[wid], src_v)
        pltpu.sync_copy(dst_hbm.at[wid], dst_v)
        plsc.subcore_barrier()

        def body(j, _):
            pltpu.async_copy(feat_hbm.at[src_v.at[j]], rows_v, sem).wait()
            pltpu.sync_copy(rows_v, acc.at[dst_v.at[j]], add=True)
            return 0

        lax.fori_loop(0, nc_w, body, 0)
        plsc.subcore_barrier()

        def wb(k, _):
            r0 = s * rpt + k * _CHUNK
            pltpu.sync_copy(acc.at[pl.ds(r0, _CHUNK)], rows_v)
            pltpu.sync_copy(rows_v, out_hbm.at[c, pl.ds(r0, _CHUNK)])
            return 0

        lax.fori_loop(0, nb, wb, 0)

    return prop


def _make_deg(nchunk):
    mesh = plsc.VectorSubcoreMesh(core_axis_name="c", subcore_axis_name="s")
    npt = _NP // _NS

    @functools.partial(
        pl.kernel,
        mesh=mesh,
        out_type=jax.ShapeDtypeStruct((_NC, _NP), jnp.float32),
        scratch_types=[
            pltpu.VMEM((nchunk, _CHUNK), jnp.int32),
            pltpu.VMEM((_CHUNK,), jnp.float32),
            pltpu.VMEM((npt,), jnp.float32),
            pltpu.VMEM_SHARED((_NP,), jnp.float32),
        ],
    )
    def degk(dst_hbm, out_hbm, dst_v, ones_v, wb_v, acc):
        c = lax.axis_index("c")
        s = lax.axis_index("s")
        wid = s * _NC + c
        zero = jnp.zeros((16,), jnp.float32)
        for k in range(_CHUNK // 16):
            ones_v[pl.ds(k * 16, 16)] = zero

        def zacc(k, _):
            pltpu.sync_copy(ones_v.at[pl.ds(0, 32)],
                            acc.at[pl.ds(s * npt + k * 32, 32)])
            return 0

        lax.fori_loop(0, npt // 32, zacc, 0)
        one = jnp.ones((16,), jnp.float32)
        for k in range(_CHUNK // 16):
            ones_v[pl.ds(k * 16, 16)] = one
        pltpu.sync_copy(dst_hbm.at[wid], dst_v)
        plsc.subcore_barrier()

        def body(j, _):
            pltpu.sync_copy(ones_v, acc.at[dst_v.at[j]], add=True)
            return 0

        lax.fori_loop(0, nchunk, body, 0)
        plsc.subcore_barrier()
        pltpu.sync_copy(acc.at[pl.ds(s * npt, npt)], wb_v)
        pltpu.sync_copy(wb_v, out_hbm.at[c, pl.ds(s * npt, npt)])

    return degk


def _s1_body(deg_ref, x_ref, dis_ref, xs_ref):
    d = deg_ref[0] + deg_ref[1] + 1.0
    dis = lax.rsqrt(d)
    dis_ref[...] = dis
    xs_ref[...] = x_ref[...] * dis


def _stage1(deg2, x_pad):
    return pl.pallas_call(
        _s1_body,
        grid=(_NP // _BLK,),
        in_specs=[
            pl.BlockSpec((2, _BLK, 1), lambda i: (0, i, 0)),
            pl.BlockSpec((_BLK, _D), lambda i: (i, 0)),
        ],
        out_specs=[
            pl.BlockSpec((_BLK, 1), lambda i: (i, 0)),
            pl.BlockSpec((_BLK, _D), lambda i: (i, 0)),
        ],
        out_shape=[
            jax.ShapeDtypeStruct((_NP, 1), jnp.float32),
            jax.ShapeDtypeStruct((_NP, _D), jnp.float32),
        ],
    )(deg2, x_pad)


def _s3_body(p1a, p1b, xs, dis, w1t, b1, w2t, hs_ref):
    a = (p1a[...] + p1b[...] + xs[...]) * dis[...]
    h = jnp.dot(a, w1t[...], preferred_element_type=jnp.float32) + b1[...]
    h = jnp.maximum(h, 0.0)
    hs_ref[...] = jnp.dot(h, w2t[...],
                          preferred_element_type=jnp.float32) * dis[...]


def _stage3(p1a, p1b, xs, dis, w1t, b1, w2t):
    hid = w1t.shape[1]
    return pl.pallas_call(
        _s3_body,
        grid=(_NP // _BLK,),
        in_specs=[
            pl.BlockSpec((_BLK, _D), lambda i: (i, 0)),
            pl.BlockSpec((_BLK, _D), lambda i: (i, 0)),
            pl.BlockSpec((_BLK, _D), lambda i: (i, 0)),
            pl.BlockSpec((_BLK, 1), lambda i: (i, 0)),
            pl.BlockSpec((_D, hid), lambda i: (0, 0)),
            pl.BlockSpec((1, hid), lambda i: (0, 0)),
            pl.BlockSpec((hid, _D), lambda i: (0, 0)),
        ],
        out_specs=pl.BlockSpec((_BLK, _D), lambda i: (i, 0)),
        out_shape=jax.ShapeDtypeStruct((_NP, _D), jnp.float32),
    )(p1a, p1b, xs, dis, w1t, b1, w2t)


def _s5_body(p2a, p2b, hs, dis, b2, out_ref):
    out_ref[...] = (p2a[...] + p2b[...] + hs[...]) * dis[...] + b2[...]


def _stage5(p2a, p2b, hs, dis, b2):
    return pl.pallas_call(
        _s5_body,
        grid=(_NP // _BLK,),
        in_specs=[
            pl.BlockSpec((_BLK, _D), lambda i: (i, 0)),
            pl.BlockSpec((_BLK, _D), lambda i: (i, 0)),
            pl.BlockSpec((_BLK, _D), lambda i: (i, 0)),
            pl.BlockSpec((_BLK, 1), lambda i: (i, 0)),
            pl.BlockSpec((1, _D), lambda i: (0, 0)),
        ],
        out_specs=pl.BlockSpec((_BLK, _D), lambda i: (i, 0)),
        out_shape=jax.ShapeDtypeStruct((_NP, _D), jnp.float32),
    )(p2a, p2b, hs, dis, b2)


def kernel(x, edge_index, W1, b1, W2, b2):
    n = x.shape[0]
    e = edge_index.shape[1]
    src = edge_index[0].astype(jnp.int32)
    dst = edge_index[1].astype(jnp.int32)

    nchunk_d = -(-e // (_NW * _CHUNK))
    pad_d = _NW * nchunk_d * _CHUNK - e
    dst_d = jnp.concatenate([dst, jnp.full((pad_d,), n, jnp.int32)])
    dst_d = dst_d.reshape(_NW, nchunk_d, _CHUNK)

    total = -(-e // _CHUNK)
    nc_b = -(-(total * 3) // (4 * _NS))
    nc_a = max(1, -(-(total - _NS * nc_b) // _NS))
    na_e, nb_e = _NS * nc_a * _CHUNK, _NS * nc_b * _CHUNK
    fillp = jnp.full((na_e + nb_e - e,), n, jnp.int32)

    def _grp(arr):
        arr = jnp.concatenate([arr, fillp])
        g0 = arr[:na_e].reshape(_NS, nc_a, _CHUNK)
        g0 = jnp.pad(g0, ((0, 0), (0, nc_b - nc_a), (0, 0)),
                     constant_values=n)
        g1 = arr[na_e:].reshape(_NS, nc_b, _CHUNK)
        return jnp.stack([g0, g1], axis=1).reshape(_NW, nc_b, _CHUNK)

    src_p = _grp(src)
    dst_p = _grp(dst)

    x_pad = jnp.pad(x, ((0, _NP - n), (0, 0)))

    deg2 = _make_deg(nchunk_d)(dst_d)
    dis, xs = _stage1(deg2.reshape(_NC, _NP, 1), x_pad)
    prop = _make_propagate(nc_a, nc_b)
    p1 = prop(xs, src_p, dst_p)
    hs = _stage3(p1[0], p1[1], xs, dis, W1.T, b1.reshape(1, -1), W2.T)
    p2 = prop(hs, src_p, dst_p)
    out = _stage5(p2[0], p2[1], hs, dis, b2.reshape(1, -1))
    return out[:n]


# speedup vs baseline: 1.1378x; 1.1378x over previous
import functools

import jax
import jax.numpy as jnp
from jax import lax
from jax.experimental import pallas as pl
from jax.experimental.pallas import tpu as pltpu
from jax.experimental.pallas import tpu_sc as plsc

_NP = 10240
_D = 128
_HD = _D // 2
_NC = 2
_NS = 16
_NW = _NC * _NS
_CHUNK = 128
_BLK = 256


def _make_propagate(nc_a, nc_b):
    nchunk = max(nc_a, nc_b)
    mesh = plsc.VectorSubcoreMesh(core_axis_name="c", subcore_axis_name="s")

    @functools.partial(
        pl.kernel,
        mesh=mesh,
        out_type=jax.ShapeDtypeStruct((_NC, _NP, _D), jnp.float32),
        scratch_types=[
            pltpu.VMEM((nchunk, _CHUNK), jnp.int32),
            pltpu.VMEM((nchunk, _CHUNK), jnp.int32),
            pltpu.VMEM((_CHUNK, _D), jnp.float32),
            pltpu.VMEM_SHARED((_NP, _D), jnp.float32),
            pltpu.SemaphoreType.DMA,
        ],
    )
    def prop(feat_hbm, src_hbm, dst_hbm, out_hbm, src_v, dst_v, rows_v, acc,
             sem):
        c = lax.axis_index("c")
        s = lax.axis_index("s")
        wid = s * _NC + c
        nc_w = jnp.where(c == 0, nc_a, nc_b)
        zero = jnp.zeros((16,), jnp.float32)

        def zrow(i, _):
            for k in range(_D // 16):
                rows_v[i, pl.ds(k * 16, 16)] = zero
            return 0

        lax.fori_loop(0, _CHUNK, zrow, 0)
        rpt = _NP // _NS
        nb = rpt // _CHUNK

        def zacc(k, _):
            pltpu.sync_copy(rows_v, acc.at[pl.ds(s * rpt + k * _CHUNK, _CHUNK)])
            return 0

        lax.fori_loop(0, nb, zacc, 0)
        pltpu.sync_copy(src_hbm.at[wid], src_v)
        pltpu.sync_copy(dst_hbm.at[wid], dst_v)
        plsc.subcore_barrier()

        def body(j, _):
            pltpu.async_copy(feat_hbm.at[src_v.at[j]], rows_v, sem).wait()
            pltpu.sync_copy(rows_v, acc.at[dst_v.at[j]], add=True)
            return 0

        lax.fori_loop(0, nc_w, body, 0)
        plsc.subcore_barrier()

        def wb(k, _):
            r0 = s * rpt + k * _CHUNK
            pltpu.sync_copy(acc.at[pl.ds(r0, _CHUNK)], rows_v)
            pltpu.sync_copy(rows_v, out_hbm.at[c, pl.ds(r0, _CHUNK)])
            return 0

        lax.fori_loop(0, nb, wb, 0)

    return prop


def _make_deg(nchunk):
    mesh = plsc.VectorSubcoreMesh(core_axis_name="c", subcore_axis_name="s")
    npt = _NP // _NS

    @functools.partial(
        pl.kernel,
        mesh=mesh,
        out_type=jax.ShapeDtypeStruct((_NC, _NP), jnp.float32),
        scratch_types=[
            pltpu.VMEM((nchunk, _CHUNK), jnp.int32),
            pltpu.VMEM((_CHUNK,), jnp.float32),
            pltpu.VMEM((npt,), jnp.float32),
            pltpu.VMEM_SHARED((_NP,), jnp.float32),
        ],
    )
    def degk(dst_hbm, out_hbm, dst_v, ones_v, wb_v, acc):
        c = lax.axis_index("c")
        s = lax.axis_index("s")
        wid = s * _NC + c
        zero = jnp.zeros((16,), jnp.float32)
        for k in range(_CHUNK // 16):
            ones_v[pl.ds(k * 16, 16)] = zero

        def zacc(k, _):
            pltpu.sync_copy(ones_v.at[pl.ds(0, 32)],
                            acc.at[pl.ds(s * npt + k * 32, 32)])
            return 0

        lax.fori_loop(0, npt // 32, zacc, 0)
        one = jnp.ones((16,), jnp.float32)
        for k in range(_CHUNK // 16):
            ones_v[pl.ds(k * 16, 16)] = one
        pltpu.sync_copy(dst_hbm.at[wid], dst_v)
        plsc.subcore_barrier()

        def body(j, _):
            pltpu.sync_copy(ones_v, acc.at[dst_v.at[j]], add=True)
            return 0

        lax.fori_loop(0, nchunk, body, 0)
        plsc.subcore_barrier()
        pltpu.sync_copy(acc.at[pl.ds(s * npt, npt)], wb_v)
        pltpu.sync_copy(wb_v, out_hbm.at[c, pl.ds(s * npt, npt)])

    return degk


def _s1_body(deg_ref, x_ref, dis_ref, xs_ref):
    d = deg_ref[0] + deg_ref[1] + 1.0
    dis = lax.rsqrt(d)
    dis_ref[...] = dis
    xs_ref[...] = x_ref[...] * dis


def _stage1(deg2, x_pad):
    return pl.pallas_call(
        _s1_body,
        grid=(_NP // _BLK,),
        in_specs=[
            pl.BlockSpec((2, _BLK, 1), lambda i: (0, i, 0)),
            pl.BlockSpec((_BLK, _D), lambda i: (i, 0)),
        ],
        out_specs=[
            pl.BlockSpec((_BLK, 1), lambda i: (i, 0)),
            pl.BlockSpec((_BLK, _D), lambda i: (i, 0)),
        ],
        out_shape=[
            jax.ShapeDtypeStruct((_NP, 1), jnp.float32),
            jax.ShapeDtypeStruct((_NP, _D), jnp.float32),
        ],
    )(deg2, x_pad)


def _s3_body(p1a, p1b, xs, dis, w1t, b1, w2t, hs_ref):
    a = (p1a[...] + p1b[...] + xs[...]) * dis[...]
    h = jnp.dot(a, w1t[...], preferred_element_type=jnp.float32) + b1[...]
    h = jnp.maximum(h, 0.0)
    hs_ref[...] = jnp.dot(h, w2t[...],
                          preferred_element_type=jnp.float32) * dis[...]


def _stage3(p1a, p1b, xs, dis, w1t, b1, w2t):
    hid = w1t.shape[1]
    return pl.pallas_call(
        _s3_body,
        grid=(_NP // _BLK,),
        in_specs=[
            pl.BlockSpec((_BLK, _D), lambda i: (i, 0)),
            pl.BlockSpec((_BLK, _D), lambda i: (i, 0)),
            pl.BlockSpec((_BLK, _D), lambda i: (i, 0)),
            pl.BlockSpec((_BLK, 1), lambda i: (i, 0)),
            pl.BlockSpec((_D, hid), lambda i: (0, 0)),
            pl.BlockSpec((1, hid), lambda i: (0, 0)),
            pl.BlockSpec((hid, _D), lambda i: (0, 0)),
        ],
        out_specs=pl.BlockSpec((_BLK, _D), lambda i: (i, 0)),
        out_shape=jax.ShapeDtypeStruct((_NP, _D), jnp.float32),
    )(p1a, p1b, xs, dis, w1t, b1, w2t)


def _s5_body(p2a, p2b, hs, dis, b2, out_ref):
    out_ref[...] = (p2a[...] + p2b[...] + hs[...]) * dis[...] + b2[...]


def _stage5(p2a, p2b, hs, dis, b2):
    return pl.pallas_call(
        _s5_body,
        grid=(_NP // _BLK,),
        in_specs=[
            pl.BlockSpec((_BLK, _D), lambda i: (i, 0)),
            pl.BlockSpec((_BLK, _D), lambda i: (i, 0)),
            pl.BlockSpec((_BLK, _D), lambda i: (i, 0)),
            pl.BlockSpec((_BLK, 1), lambda i: (i, 0)),
            pl.BlockSpec((1, _D), lambda i: (0, 0)),
        ],
        out_specs=pl.BlockSpec((_BLK, _D), lambda i: (i, 0)),
        out_shape=jax.ShapeDtypeStruct((_NP, _D), jnp.float32),
    )(p2a, p2b, hs, dis, b2)


def kernel(x, edge_index, W1, b1, W2, b2):
    n = x.shape[0]
    e = edge_index.shape[1]
    src = edge_index[0].astype(jnp.int32)
    dst = edge_index[1].astype(jnp.int32)

    nchunk_d = -(-e // (_NW * _CHUNK))
    pad_d = _NW * nchunk_d * _CHUNK - e
    dst_d = jnp.concatenate([dst, jnp.full((pad_d,), n, jnp.int32)])
    dst_d = dst_d.reshape(_NW, nchunk_d, _CHUNK)

    total = -(-e // _CHUNK)
    nc_b = -(-(total * 3) // (5 * _NS))
    nc_a = max(1, -(-(total - _NS * nc_b) // _NS))
    na_e, nb_e = _NS * nc_a * _CHUNK, _NS * nc_b * _CHUNK
    fillp = jnp.full((na_e + nb_e - e,), n, jnp.int32)

    def _grp(arr):
        arr = jnp.concatenate([arr, fillp])
        g0 = arr[:na_e].reshape(_NS, nc_a, _CHUNK)
        g0 = jnp.pad(g0, ((0, 0), (0, nc_b - nc_a), (0, 0)),
                     constant_values=n)
        g1 = arr[na_e:].reshape(_NS, nc_b, _CHUNK)
        return jnp.stack([g0, g1], axis=1).reshape(_NW, nc_b, _CHUNK)

    src_p = _grp(src)
    dst_p = _grp(dst)

    x_pad = jnp.pad(x, ((0, _NP - n), (0, 0)))

    deg2 = _make_deg(nchunk_d)(dst_d)
    dis, xs = _stage1(deg2.reshape(_NC, _NP, 1), x_pad)
    prop = _make_propagate(nc_a, nc_b)
    p1 = prop(xs, src_p, dst_p)
    hs = _stage3(p1[0], p1[1], xs, dis, W1.T, b1.reshape(1, -1), W2.T)
    p2 = prop(hs, src_p, dst_p)
    out = _stage5(p2[0], p2[1], hs, dis, b2.reshape(1, -1))
    return out[:n]


# speedup vs baseline: 1.1442x; 1.0056x over previous
import functools

import jax
import jax.numpy as jnp
from jax import lax
from jax.experimental import pallas as pl
from jax.experimental.pallas import tpu as pltpu
from jax.experimental.pallas import tpu_sc as plsc

_NP = 10240
_D = 128
_HD = _D // 2
_NC = 2
_NS = 16
_NW = _NC * _NS
_CHUNK = 128
_BLK = 256


def _make_propagate(nc_a, nc_b):
    nchunk = max(nc_a, nc_b)
    mesh = plsc.VectorSubcoreMesh(core_axis_name="c", subcore_axis_name="s")

    @functools.partial(
        pl.kernel,
        mesh=mesh,
        out_type=jax.ShapeDtypeStruct((_NC, _NP, _D), jnp.float32),
        scratch_types=[
            pltpu.VMEM((nchunk, _CHUNK), jnp.int32),
            pltpu.VMEM((nchunk, _CHUNK), jnp.int32),
            pltpu.VMEM((_CHUNK, _D), jnp.float32),
            pltpu.VMEM_SHARED((_NP, _D), jnp.float32),
            pltpu.SemaphoreType.DMA,
        ],
    )
    def prop(feat_hbm, src_hbm, dst_hbm, out_hbm, src_v, dst_v, rows_v, acc,
             sem):
        c = lax.axis_index("c")
        s = lax.axis_index("s")
        wid = s * _NC + c
        nc_w = jnp.where(c == 0, nc_a, nc_b)
        zero = jnp.zeros((16,), jnp.float32)

        def zrow(i, _):
            for k in range(_D // 16):
                rows_v[i, pl.ds(k * 16, 16)] = zero
            return 0

        lax.fori_loop(0, _CHUNK, zrow, 0)
        rpt = _NP // _NS
        nb = rpt // _CHUNK

        def zacc(k, _):
            pltpu.sync_copy(rows_v, acc.at[pl.ds(s * rpt + k * _CHUNK, _CHUNK)])
            return 0

        lax.fori_loop(0, nb, zacc, 0)
        pltpu.sync_copy(src_hbm.at[wid], src_v)
        pltpu.sync_copy(dst_hbm.at[wid], dst_v)
        plsc.subcore_barrier()

        def body(j, _):
            pltpu.async_copy(feat_hbm.at[src_v.at[j]], rows_v, sem).wait()
            pltpu.sync_copy(rows_v, acc.at[dst_v.at[j]], add=True)
            return 0

        lax.fori_loop(0, nc_w, body, 0)
        plsc.subcore_barrier()

        def wb(k, _):
            r0 = s * rpt + k * _CHUNK
            pltpu.sync_copy(acc.at[pl.ds(r0, _CHUNK)], rows_v)
            pltpu.sync_copy(rows_v, out_hbm.at[c, pl.ds(r0, _CHUNK)])
            return 0

        lax.fori_loop(0, nb, wb, 0)

    return prop


def _make_deg(nchunk):
    mesh = plsc.VectorSubcoreMesh(core_axis_name="c", subcore_axis_name="s")
    npt = _NP // _NS

    @functools.partial(
        pl.kernel,
        mesh=mesh,
        out_type=jax.ShapeDtypeStruct((_NC, _NP), jnp.float32),
        scratch_types=[
            pltpu.VMEM((nchunk, _CHUNK), jnp.int32),
            pltpu.VMEM((_CHUNK,), jnp.float32),
            pltpu.VMEM((npt,), jnp.float32),
            pltpu.VMEM_SHARED((_NP,), jnp.float32),
        ],
    )
    def degk(dst_hbm, out_hbm, dst_v, ones_v, wb_v, acc):
        c = lax.axis_index("c")
        s = lax.axis_index("s")
        wid = s * _NC + c
        zero = jnp.zeros((16,), jnp.float32)
        for k in range(_CHUNK // 16):
            ones_v[pl.ds(k * 16, 16)] = zero

        def zacc(k, _):
            pltpu.sync_copy(ones_v.at[pl.ds(0, 32)],
                            acc.at[pl.ds(s * npt + k * 32, 32)])
            return 0

        lax.fori_loop(0, npt // 32, zacc, 0)
        one = jnp.ones((16,), jnp.float32)
        for k in range(_CHUNK // 16):
            ones_v[pl.ds(k * 16, 16)] = one
        pltpu.sync_copy(dst_hbm.at[wid], dst_v)
        plsc.subcore_barrier()

        def body(j, _):
            pltpu.sync_copy(ones_v, acc.at[dst_v.at[j]], add=True)
            return 0

        lax.fori_loop(0, nchunk, body, 0)
        plsc.subcore_barrier()
        pltpu.sync_copy(acc.at[pl.ds(s * npt, npt)], wb_v)
        pltpu.sync_copy(wb_v, out_hbm.at[c, pl.ds(s * npt, npt)])

    return degk


def _s1_body(deg_ref, x_ref, dis_ref, xs_ref):
    d = deg_ref[0] + deg_ref[1] + 1.0
    dis = lax.rsqrt(d)
    dis_ref[...] = dis
    xs_ref[...] = x_ref[...] * dis


def _stage1(deg2, x_pad):
    return pl.pallas_call(
        _s1_body,
        grid=(_NP // _BLK,),
        in_specs=[
            pl.BlockSpec((2, _BLK, 1), lambda i: (0, i, 0)),
            pl.BlockSpec((_BLK, _D), lambda i: (i, 0)),
        ],
        out_specs=[
            pl.BlockSpec((_BLK, 1), lambda i: (i, 0)),
            pl.BlockSpec((_BLK, _D), lambda i: (i, 0)),
        ],
        out_shape=[
            jax.ShapeDtypeStruct((_NP, 1), jnp.float32),
            jax.ShapeDtypeStruct((_NP, _D), jnp.float32),
        ],
    )(deg2, x_pad)


def _s3_body(p1a, p1b, xs, dis, w1t, b1, w2t, hs_ref):
    a = (p1a[...] + p1b[...] + xs[...]) * dis[...]
    h = jnp.dot(a, w1t[...], preferred_element_type=jnp.float32) + b1[...]
    h = jnp.maximum(h, 0.0)
    hs_ref[...] = jnp.dot(h, w2t[...],
                          preferred_element_type=jnp.float32) * dis[...]


def _stage3(p1a, p1b, xs, dis, w1t, b1, w2t):
    hid = w1t.shape[1]
    return pl.pallas_call(
        _s3_body,
        grid=(_NP // _BLK,),
        in_specs=[
            pl.BlockSpec((_BLK, _D), lambda i: (i, 0)),
            pl.BlockSpec((_BLK, _D), lambda i: (i, 0)),
            pl.BlockSpec((_BLK, _D), lambda i: (i, 0)),
            pl.BlockSpec((_BLK, 1), lambda i: (i, 0)),
            pl.BlockSpec((_D, hid), lambda i: (0, 0)),
            pl.BlockSpec((1, hid), lambda i: (0, 0)),
            pl.BlockSpec((hid, _D), lambda i: (0, 0)),
        ],
        out_specs=pl.BlockSpec((_BLK, _D), lambda i: (i, 0)),
        out_shape=jax.ShapeDtypeStruct((_NP, _D), jnp.float32),
    )(p1a, p1b, xs, dis, w1t, b1, w2t)


def _s5_body(p2a, p2b, hs, dis, b2, out_ref):
    out_ref[...] = (p2a[...] + p2b[...] + hs[...]) * dis[...] + b2[...]


def _stage5(p2a, p2b, hs, dis, b2):
    return pl.pallas_call(
        _s5_body,
        grid=(_NP // _BLK,),
        in_specs=[
            pl.BlockSpec((_BLK, _D), lambda i: (i, 0)),
            pl.BlockSpec((_BLK, _D), lambda i: (i, 0)),
            pl.BlockSpec((_BLK, _D), lambda i: (i, 0)),
            pl.BlockSpec((_BLK, 1), lambda i: (i, 0)),
            pl.BlockSpec((1, _D), lambda i: (0, 0)),
        ],
        out_specs=pl.BlockSpec((_BLK, _D), lambda i: (i, 0)),
        out_shape=jax.ShapeDtypeStruct((_NP, _D), jnp.float32),
    )(p2a, p2b, hs, dis, b2)


def kernel(x, edge_index, W1, b1, W2, b2):
    n = x.shape[0]
    e = edge_index.shape[1]
    src = edge_index[0].astype(jnp.int32)
    dst = edge_index[1].astype(jnp.int32)

    nchunk_d = -(-e // (_NW * _CHUNK))
    pad_d = _NW * nchunk_d * _CHUNK - e
    dst_d = jnp.concatenate([dst, jnp.full((pad_d,), n, jnp.int32)])
    dst_d = dst_d.reshape(_NW, nchunk_d, _CHUNK)

    total = -(-e // _CHUNK)
    nc_b = -(-(total * 11) // (20 * _NS))
    nc_a = max(1, -(-(total - _NS * nc_b) // _NS))
    na_e, nb_e = _NS * nc_a * _CHUNK, _NS * nc_b * _CHUNK
    fillp = jnp.full((na_e + nb_e - e,), n, jnp.int32)

    def _grp(arr):
        arr = jnp.concatenate([arr, fillp])
        g0 = arr[:na_e].reshape(_NS, nc_a, _CHUNK)
        g0 = jnp.pad(g0, ((0, 0), (0, nc_b - nc_a), (0, 0)),
                     constant_values=n)
        g1 = arr[na_e:].reshape(_NS, nc_b, _CHUNK)
        return jnp.stack([g0, g1], axis=1).reshape(_NW, nc_b, _CHUNK)

    src_p = _grp(src)
    dst_p = _grp(dst)

    x_pad = jnp.pad(x, ((0, _NP - n), (0, 0)))

    deg2 = _make_deg(nchunk_d)(dst_d)
    dis, xs = _stage1(deg2.reshape(_NC, _NP, 1), x_pad)
    prop = _make_propagate(nc_a, nc_b)
    p1 = prop(xs, src_p, dst_p)
    hs = _stage3(p1[0], p1[1], xs, dis, W1.T, b1.reshape(1, -1), W2.T)
    p2 = prop(hs, src_p, dst_p)
    out = _stage5(p2[0], p2[1], hs, dis, b2.reshape(1, -1))
    return out[:n]
